# Initial kernel scaffold; baseline (speedup 1.0000x reference)
import jax
import jax.numpy as jnp
from jax.experimental import pallas as pl


def kernel(img, gts, segments, n_tokens, W1, W2, Wq, Wk, Wv, Wo):
    raise NotImplementedError("write your pallas kernel here")


# speedup vs baseline: 9.2048x; 9.2048x over previous
import jax
import jax.numpy as jnp
from jax.experimental import pallas as pl
from jax.experimental.pallas import tpu as pltpu

B, H, W_ = 4, 384, 384
CIN, CF, NCLS = 3, 96, 10
MAXLEN = 512
NTOK = 512
N = H * W_
P = 2048
NT = N // P
NMETA = 16
NR = CF + NMETA


def _conv_seg_kernel(off_ref, x_ref, seg_ref, gts_ref, w1_ref, w2_ref,
                     cnn_ref, segg_ref, acc_ref):
    t = pl.program_id(1)
    x = x_ref[0]
    seg = seg_ref[0]
    g = gts_ref[0]
    feats = jax.nn.relu(
        jax.lax.dot_general(w1_ref[...], x, (((0,), (0,)), ((), ()))))
    cnn_ref[0] = jax.lax.dot_general(
        w2_ref[...], feats, (((0,), (0,)), ((), ())))
    segg_ref[0] = seg + off_ref[0]
    m = (jax.lax.broadcasted_iota(jnp.int32, (NTOK, P), 0)
         == seg).astype(jnp.float32)
    mi = jax.lax.broadcasted_iota(jnp.int32, (NMETA, P), 0)
    oh = jnp.logical_or(mi == g, mi == NCLS).astype(jnp.float32)
    rhs = jnp.concatenate([feats, oh], axis=0)
    upd = jax.lax.dot_general(rhs, m, (((1,), (1,)), ((), ())))

    @pl.when(t == 0)
    def _init():
        acc_ref[0] = jnp.zeros_like(acc_ref[0])

    acc_ref[0] += upd


def _attn_kernel(nt_ref, acc_ref, wq_ref, wk_ref, wv_ref, wo_ref,
                 tl_ref, lab_ref, mask_ref):
    acc = acc_ref[0]
    counts = acc[CF + NCLS:CF + NCLS + 1, :]
    tokens_t = acc[:CF] / jnp.maximum(counts, 1.0)
    clc = acc[CF:CF + NCLS]
    mx = jnp.max(clc, axis=0, keepdims=True)
    idxv = jax.lax.broadcasted_iota(jnp.int32, (NCLS, NTOK), 0).astype(jnp.float32)
    lab_ref[0] = jnp.min(jnp.where(clc == mx, idxv, jnp.float32(NCLS)),
                         axis=0, keepdims=True)
    n = nt_ref[0]
    valid_row = (jax.lax.broadcasted_iota(jnp.int32, (1, MAXLEN), 1)
                 < n).astype(jnp.float32)
    valid_col = (jax.lax.broadcasted_iota(jnp.int32, (MAXLEN, 1), 0)
                 < n).astype(jnp.float32)
    mask_ref[0] = valid_row

    cdims = (((0,), (0,)), ((), ()))
    q = jax.lax.dot_general(tokens_t, wq_ref[...], cdims)
    k = jax.lax.dot_general(tokens_t, wk_ref[...], cdims)
    v = jax.lax.dot_general(tokens_t, wv_ref[...], cdims)
    scores = jax.lax.dot_general(
        q, k, (((1,), (1,)), ((), ()))) / jnp.sqrt(jnp.float32(CF))
    smax = jnp.max(scores, axis=1, keepdims=True)
    e = jnp.exp(scores - smax)
    attn = e / jnp.sum(e, axis=1, keepdims=True)
    attn = attn * valid_row * valid_col
    ctx = jax.lax.dot_general(attn, v, (((1,), (0,)), ((), ())))
    tl_ref[0] = jax.lax.dot_general(ctx, wo_ref[...], (((1,), (0,)), ((), ())))


def kernel(img, gts, segments, n_tokens, W1, W2, Wq, Wk, Wv, Wo):
    x = img.reshape(B, CIN, N)
    seg3 = segments.reshape(B, 1, N)
    gts3 = gts.reshape(B, 1, N)
    offsets = jnp.concatenate(
        [jnp.zeros((1,), dtype=n_tokens.dtype), jnp.cumsum(n_tokens)[:-1]])
    off1 = (offsets + 1).astype(jnp.int32).reshape(B, 1, 1)

    cnn_flat, seg_global_flat, acc = pl.pallas_call(
        _conv_seg_kernel,
        grid=(B, NT),
        in_specs=[
            pl.BlockSpec((1, 1, 1), lambda b, t: (b, 0, 0)),
            pl.BlockSpec((1, CIN, P), lambda b, t: (b, 0, t)),
            pl.BlockSpec((1, 1, P), lambda b, t: (b, 0, t)),
            pl.BlockSpec((1, 1, P), lambda b, t: (b, 0, t)),
            pl.BlockSpec((CIN, CF), lambda b, t: (0, 0)),
            pl.BlockSpec((CF, NCLS), lambda b, t: (0, 0)),
        ],
        out_specs=[
            pl.BlockSpec((1, NCLS, P), lambda b, t: (b, 0, t)),
            pl.BlockSpec((1, 1, P), lambda b, t: (b, 0, t)),
            pl.BlockSpec((1, NR, NTOK), lambda b, t: (b, 0, 0)),
        ],
        out_shape=[
            jax.ShapeDtypeStruct((B, NCLS, N), jnp.float32),
            jax.ShapeDtypeStruct((B, 1, N), jnp.int32),
            jax.ShapeDtypeStruct((B, NR, NTOK), jnp.float32),
        ],
    )(off1, x, seg3, gts3, W1, W2)

    nt3 = n_tokens.astype(jnp.int32).reshape(B, 1, 1)
    trans_logits, super_labels, mask = pl.pallas_call(
        _attn_kernel,
        grid=(B,),
        in_specs=[
            pl.BlockSpec((1, 1, 1), lambda b: (b, 0, 0)),
            pl.BlockSpec((1, NR, NTOK), lambda b: (b, 0, 0)),
            pl.BlockSpec((CF, CF), lambda b: (0, 0)),
            pl.BlockSpec((CF, CF), lambda b: (0, 0)),
            pl.BlockSpec((CF, CF), lambda b: (0, 0)),
            pl.BlockSpec((CF, NCLS), lambda b: (0, 0)),
        ],
        out_specs=[
            pl.BlockSpec((1, MAXLEN, NCLS), lambda b: (b, 0, 0)),
            pl.BlockSpec((1, 1, MAXLEN), lambda b: (b, 0, 0)),
            pl.BlockSpec((1, 1, MAXLEN), lambda b: (b, 0, 0)),
        ],
        out_shape=[
            jax.ShapeDtypeStruct((B, MAXLEN, NCLS), jnp.float32),
            jax.ShapeDtypeStruct((B, 1, MAXLEN), jnp.float32),
            jax.ShapeDtypeStruct((B, 1, MAXLEN), jnp.float32),
        ],
    )(nt3, acc, Wq, Wk, Wv, Wo)

    cnn_logits = cnn_flat.reshape(B, NCLS, H, W_)
    seg_global = seg_global_flat.reshape(B, H, W_)
    tokens_ids = jnp.arange(1, B * NTOK + 1, dtype=jnp.int32)
    return (cnn_logits, trans_logits, super_labels.reshape(B, MAXLEN),
            mask.reshape(B, MAXLEN), tokens_ids, seg_global)


# speedup vs baseline: 10.8155x; 1.1750x over previous
import jax
import jax.numpy as jnp
from jax.experimental import pallas as pl
from jax.experimental.pallas import tpu as pltpu

B, H, W_ = 4, 384, 384
CIN, CF, NCLS = 3, 96, 10
MAXLEN = 512
NTOK = 512
N = H * W_
P = 4096
NT = N // P
NMETA = 16
NR = CF + NMETA


def _conv_seg_kernel(off_ref, x_ref, seg_ref, gts_ref, w1_ref, w2_ref,
                     cnn_ref, segg_ref, acc_ref):
    t = pl.program_id(1)
    x = x_ref[0]
    seg = seg_ref[0]
    g = gts_ref[0]
    feats = jax.nn.relu(
        jax.lax.dot_general(w1_ref[...], x, (((0,), (0,)), ((), ()))))
    cnn_ref[0] = jax.lax.dot_general(
        w2_ref[...], feats, (((0,), (0,)), ((), ())))
    segg_ref[0] = seg + off_ref[0]
    m = (jax.lax.broadcasted_iota(jnp.int32, (NTOK, P), 0)
         == seg).astype(jnp.bfloat16)
    mi = jax.lax.broadcasted_iota(jnp.int32, (NMETA, P), 0)
    oh = jnp.logical_or(mi == g, mi == NCLS).astype(jnp.bfloat16)
    rhs = jnp.concatenate([feats.astype(jnp.bfloat16), oh], axis=0)
    upd = jax.lax.dot_general(rhs, m, (((1,), (1,)), ((), ())),
                              preferred_element_type=jnp.float32)

    @pl.when(t == 0)
    def _init():
        acc_ref[0] = jnp.zeros_like(acc_ref[0])

    acc_ref[0] += upd


def _attn_kernel(nt_ref, acc_ref, wq_ref, wk_ref, wv_ref, wo_ref,
                 tl_ref, lab_ref, mask_ref):
    acc = acc_ref[0]
    counts = acc[CF + NCLS:CF + NCLS + 1, :]
    tokens_t = acc[:CF] / jnp.maximum(counts, 1.0)
    clc = acc[CF:CF + NCLS]
    mx = jnp.max(clc, axis=0, keepdims=True)
    idxv = jax.lax.broadcasted_iota(jnp.int32, (NCLS, NTOK), 0).astype(jnp.float32)
    lab_ref[0] = jnp.min(jnp.where(clc == mx, idxv, jnp.float32(NCLS)),
                         axis=0, keepdims=True)
    n = nt_ref[0]
    valid_row = (jax.lax.broadcasted_iota(jnp.int32, (1, MAXLEN), 1)
                 < n).astype(jnp.float32)
    valid_col = (jax.lax.broadcasted_iota(jnp.int32, (MAXLEN, 1), 0)
                 < n).astype(jnp.float32)
    mask_ref[0] = valid_row

    cdims = (((0,), (0,)), ((), ()))
    q = jax.lax.dot_general(tokens_t, wq_ref[...], cdims)
    k = jax.lax.dot_general(tokens_t, wk_ref[...], cdims)
    v = jax.lax.dot_general(tokens_t, wv_ref[...], cdims)
    scores = jax.lax.dot_general(
        q, k, (((1,), (1,)), ((), ()))) / jnp.sqrt(jnp.float32(CF))
    smax = jnp.max(scores, axis=1, keepdims=True)
    e = jnp.exp(scores - smax)
    attn = e / jnp.sum(e, axis=1, keepdims=True)
    attn = attn * valid_row * valid_col
    ctx = jax.lax.dot_general(attn, v, (((1,), (0,)), ((), ())))
    tl_ref[0] = jax.lax.dot_general(ctx, wo_ref[...], (((1,), (0,)), ((), ())))


def kernel(img, gts, segments, n_tokens, W1, W2, Wq, Wk, Wv, Wo):
    x = img.reshape(B, CIN, N)
    seg3 = segments.reshape(B, 1, N)
    gts3 = gts.reshape(B, 1, N)
    offsets = jnp.concatenate(
        [jnp.zeros((1,), dtype=n_tokens.dtype), jnp.cumsum(n_tokens)[:-1]])
    off1 = (offsets + 1).astype(jnp.int32).reshape(B, 1, 1)

    cnn_flat, seg_global_flat, acc = pl.pallas_call(
        _conv_seg_kernel,
        grid=(B, NT),
        in_specs=[
            pl.BlockSpec((1, 1, 1), lambda b, t: (b, 0, 0)),
            pl.BlockSpec((1, CIN, P), lambda b, t: (b, 0, t)),
            pl.BlockSpec((1, 1, P), lambda b, t: (b, 0, t)),
            pl.BlockSpec((1, 1, P), lambda b, t: (b, 0, t)),
            pl.BlockSpec((CIN, CF), lambda b, t: (0, 0)),
            pl.BlockSpec((CF, NCLS), lambda b, t: (0, 0)),
        ],
        out_specs=[
            pl.BlockSpec((1, NCLS, P), lambda b, t: (b, 0, t)),
            pl.BlockSpec((1, 1, P), lambda b, t: (b, 0, t)),
            pl.BlockSpec((1, NR, NTOK), lambda b, t: (b, 0, 0)),
        ],
        out_shape=[
            jax.ShapeDtypeStruct((B, NCLS, N), jnp.float32),
            jax.ShapeDtypeStruct((B, 1, N), jnp.int32),
            jax.ShapeDtypeStruct((B, NR, NTOK), jnp.float32),
        ],
    )(off1, x, seg3, gts3, W1, W2)

    nt3 = n_tokens.astype(jnp.int32).reshape(B, 1, 1)
    trans_logits, super_labels, mask = pl.pallas_call(
        _attn_kernel,
        grid=(B,),
        in_specs=[
            pl.BlockSpec((1, 1, 1), lambda b: (b, 0, 0)),
            pl.BlockSpec((1, NR, NTOK), lambda b: (b, 0, 0)),
            pl.BlockSpec((CF, CF), lambda b: (0, 0)),
            pl.BlockSpec((CF, CF), lambda b: (0, 0)),
            pl.BlockSpec((CF, CF), lambda b: (0, 0)),
            pl.BlockSpec((CF, NCLS), lambda b: (0, 0)),
        ],
        out_specs=[
            pl.BlockSpec((1, MAXLEN, NCLS), lambda b: (b, 0, 0)),
            pl.BlockSpec((1, 1, MAXLEN), lambda b: (b, 0, 0)),
            pl.BlockSpec((1, 1, MAXLEN), lambda b: (b, 0, 0)),
        ],
        out_shape=[
            jax.ShapeDtypeStruct((B, MAXLEN, NCLS), jnp.float32),
            jax.ShapeDtypeStruct((B, 1, MAXLEN), jnp.float32),
            jax.ShapeDtypeStruct((B, 1, MAXLEN), jnp.float32),
        ],
    )(nt3, acc, Wq, Wk, Wv, Wo)

    cnn_logits = cnn_flat.reshape(B, NCLS, H, W_)
    seg_global = seg_global_flat.reshape(B, H, W_)
    tokens_ids = jnp.arange(1, B * NTOK + 1, dtype=jnp.int32)
    return (cnn_logits, trans_logits, super_labels.reshape(B, MAXLEN),
            mask.reshape(B, MAXLEN), tokens_ids, seg_global)


# speedup vs baseline: 11.4282x; 1.0567x over previous
import jax
import jax.numpy as jnp
from jax.experimental import pallas as pl
from jax.experimental.pallas import tpu as pltpu

B, H, W_ = 4, 384, 384
CIN, CF, NCLS = 3, 96, 10
MAXLEN = 512
NTOK = 512
N = H * W_
P = 8192
NT = N // P
NMETA = 16
NR = CF + NMETA


def _conv_seg_kernel(off_ref, x_ref, seg_ref, gts_ref, w1_ref, w2_ref,
                     cnn_ref, segg_ref, acc_ref):
    t = pl.program_id(1)
    x = x_ref[0]
    seg = seg_ref[0]
    g = gts_ref[0]
    feats = jax.nn.relu(
        jax.lax.dot_general(w1_ref[...], x, (((0,), (0,)), ((), ()))))
    cnn_ref[0] = jax.lax.dot_general(
        w2_ref[...], feats, (((0,), (0,)), ((), ())))
    segg_ref[0] = seg + off_ref[0]
    mi = jax.lax.broadcasted_iota(jnp.int32, (NMETA, P), 0)
    oh = jnp.logical_or(mi == g, mi == NCLS).astype(jnp.bfloat16)
    rhs = jnp.concatenate([feats.astype(jnp.bfloat16), oh], axis=0)

    @pl.when(t == 0)
    def _init():
        acc_ref[0] = jnp.zeros_like(acc_ref[0])

    m = (jax.lax.broadcasted_iota(jnp.int32, (NTOK, P), 0)
         == seg).astype(jnp.bfloat16)
    upd = jax.lax.dot_general(rhs, m, (((1,), (1,)), ((), ())),
                              preferred_element_type=jnp.float32)
    acc_ref[0] += upd


def _attn_kernel(nt_ref, acc_ref, wq_ref, wk_ref, wv_ref, wo_ref,
                 tl_ref, lab_ref, mask_ref):
    acc = acc_ref[0]
    counts = acc[CF + NCLS:CF + NCLS + 1, :]
    tokens_t = acc[:CF] / jnp.maximum(counts, 1.0)
    clc = acc[CF:CF + NCLS]
    mx = jnp.max(clc, axis=0, keepdims=True)
    idxv = jax.lax.broadcasted_iota(jnp.int32, (NCLS, NTOK), 0).astype(jnp.float32)
    lab_ref[0] = jnp.min(jnp.where(clc == mx, idxv, jnp.float32(NCLS)),
                         axis=0, keepdims=True)
    n = nt_ref[0]
    valid_row = (jax.lax.broadcasted_iota(jnp.int32, (1, MAXLEN), 1)
                 < n).astype(jnp.float32)
    valid_col = (jax.lax.broadcasted_iota(jnp.int32, (MAXLEN, 1), 0)
                 < n).astype(jnp.float32)
    mask_ref[0] = valid_row

    cdims = (((0,), (0,)), ((), ()))
    q = jax.lax.dot_general(tokens_t, wq_ref[...], cdims)
    k = jax.lax.dot_general(tokens_t, wk_ref[...], cdims)
    v = jax.lax.dot_general(tokens_t, wv_ref[...], cdims)
    scores = jax.lax.dot_general(
        q, k, (((1,), (1,)), ((), ()))) / jnp.sqrt(jnp.float32(CF))
    smax = jnp.max(scores, axis=1, keepdims=True)
    e = jnp.exp(scores - smax)
    attn = e / jnp.sum(e, axis=1, keepdims=True)
    attn = attn * valid_row * valid_col
    ctx = jax.lax.dot_general(attn, v, (((1,), (0,)), ((), ())))
    tl_ref[0] = jax.lax.dot_general(ctx, wo_ref[...], (((1,), (0,)), ((), ())))


def kernel(img, gts, segments, n_tokens, W1, W2, Wq, Wk, Wv, Wo):
    x = img.reshape(B, CIN, N)
    seg3 = segments.reshape(B, 1, N)
    gts3 = gts.reshape(B, 1, N)
    offsets = jnp.concatenate(
        [jnp.zeros((1,), dtype=n_tokens.dtype), jnp.cumsum(n_tokens)[:-1]])
    off1 = (offsets + 1).astype(jnp.int32).reshape(B, 1, 1)

    cnn_flat, seg_global_flat, acc = pl.pallas_call(
        _conv_seg_kernel,
        grid=(B, NT),
        in_specs=[
            pl.BlockSpec((1, 1, 1), lambda b, t: (b, 0, 0)),
            pl.BlockSpec((1, CIN, P), lambda b, t: (b, 0, t)),
            pl.BlockSpec((1, 1, P), lambda b, t: (b, 0, t)),
            pl.BlockSpec((1, 1, P), lambda b, t: (b, 0, t)),
            pl.BlockSpec((CIN, CF), lambda b, t: (0, 0)),
            pl.BlockSpec((CF, NCLS), lambda b, t: (0, 0)),
        ],
        out_specs=[
            pl.BlockSpec((1, NCLS, P), lambda b, t: (b, 0, t)),
            pl.BlockSpec((1, 1, P), lambda b, t: (b, 0, t)),
            pl.BlockSpec((1, NR, NTOK), lambda b, t: (b, 0, 0)),
        ],
        out_shape=[
            jax.ShapeDtypeStruct((B, NCLS, N), jnp.float32),
            jax.ShapeDtypeStruct((B, 1, N), jnp.int32),
            jax.ShapeDtypeStruct((B, NR, NTOK), jnp.float32),
        ],
    )(off1, x, seg3, gts3, W1, W2)

    nt3 = n_tokens.astype(jnp.int32).reshape(B, 1, 1)
    trans_logits, super_labels, mask = pl.pallas_call(
        _attn_kernel,
        grid=(B,),
        in_specs=[
            pl.BlockSpec((1, 1, 1), lambda b: (b, 0, 0)),
            pl.BlockSpec((1, NR, NTOK), lambda b: (b, 0, 0)),
            pl.BlockSpec((CF, CF), lambda b: (0, 0)),
            pl.BlockSpec((CF, CF), lambda b: (0, 0)),
            pl.BlockSpec((CF, CF), lambda b: (0, 0)),
            pl.BlockSpec((CF, NCLS), lambda b: (0, 0)),
        ],
        out_specs=[
            pl.BlockSpec((1, MAXLEN, NCLS), lambda b: (b, 0, 0)),
            pl.BlockSpec((1, 1, MAXLEN), lambda b: (b, 0, 0)),
            pl.BlockSpec((1, 1, MAXLEN), lambda b: (b, 0, 0)),
        ],
        out_shape=[
            jax.ShapeDtypeStruct((B, MAXLEN, NCLS), jnp.float32),
            jax.ShapeDtypeStruct((B, 1, MAXLEN), jnp.float32),
            jax.ShapeDtypeStruct((B, 1, MAXLEN), jnp.float32),
        ],
    )(nt3, acc, Wq, Wk, Wv, Wo)

    cnn_logits = cnn_flat.reshape(B, NCLS, H, W_)
    seg_global = seg_global_flat.reshape(B, H, W_)
    tokens_ids = jnp.arange(1, B * NTOK + 1, dtype=jnp.int32)
    return (cnn_logits, trans_logits, super_labels.reshape(B, MAXLEN),
            mask.reshape(B, MAXLEN), tokens_ids, seg_global)
